# Initial kernel scaffold; baseline (speedup 1.0000x reference)
import functools

import jax
import jax.numpy as jnp
from jax import lax
from jax.experimental import pallas as pl
from jax.experimental.pallas import tpu as pltpu
from jax.experimental.pallas import tpu_sc as plsc

N_NODES = 10000
N_EDGES = 320000
F = 128

NC = 2
NS = 16
NW = NC * NS

CHUNK = 128
CH_PER_TILE = 80
E_PAD = NW * CH_PER_TILE * CHUNK
N_PAD = 10240
ROWS_PER_TILE = N_PAD // NS
DUMMY_COL = N_NODES

_mesh = plsc.VectorSubcoreMesh(
    core_axis_name="c", subcore_axis_name="s", num_cores=NC, num_subcores=NS)


def _newton_rsqrt(d):
    i = plsc.bitcast(d, jnp.int32)
    i = jnp.int32(0x5F3759DF) - lax.shift_right_logical(i, 1)
    y = plsc.bitcast(i, jnp.float32)
    half = d * jnp.float32(0.5)
    for _ in range(3):
        y = y * (jnp.float32(1.5) - half * y * y)
    return y


def _deg_isd_body(col2d_hbm, zeros1_hbm, isd_hbm, cidx_v, ones_v, deg_v, isd_v,
                  deg_acc):
    cid = lax.axis_index("c")
    sid = lax.axis_index("s")

    @pl.when(cid == 0)
    def _():
        pltpu.sync_copy(zeros1_hbm, deg_acc.at[pl.ds(sid * ROWS_PER_TILE,
                                                     ROWS_PER_TILE)])
        for j in range(CHUNK // 16):
            ones_v[pl.ds(j * 16, 16)] = jnp.ones((16,), jnp.float32)
        plsc.subcore_barrier()

        rows = (E_PAD // CHUNK) // NS
        pltpu.sync_copy(col2d_hbm.at[pl.ds(sid * rows, rows)], cidx_v)

        def body(k, carry):
            pltpu.sync_copy(ones_v, deg_acc.at[cidx_v.at[k]], add=True)
            return carry
        lax.fori_loop(0, rows, body, 0)
        plsc.subcore_barrier()

        base = sid * ROWS_PER_TILE
        pltpu.sync_copy(deg_acc.at[pl.ds(base, ROWS_PER_TILE)], deg_v)
        for j in range(ROWS_PER_TILE // 16):
            d = jnp.maximum(deg_v[pl.ds(j * 16, 16)], jnp.float32(1.0))
            isd_v[pl.ds(j * 16, 16)] = _newton_rsqrt(d)
        pltpu.sync_copy(isd_v, isd_hbm.at[pl.ds(base, ROWS_PER_TILE)])


_deg_isd = functools.partial(
    pl.kernel, _deg_isd_body, mesh=_mesh,
    out_type=jax.ShapeDtypeStruct((N_PAD,), jnp.float32),
    scratch_types=[
        pltpu.VMEM(((E_PAD // CHUNK) // NS, CHUNK), jnp.int32),
        pltpu.VMEM((CHUNK,), jnp.float32),
        pltpu.VMEM((ROWS_PER_TILE,), jnp.float32),
        pltpu.VMEM((ROWS_PER_TILE,), jnp.float32),
        pltpu.VMEM_SHARED((N_PAD,), jnp.float32),
    ],
)()


def _spmm_body(row2d_hbm, col2d_hbm, hp_hbm, zeros2_hbm, part_hbm,
               ridx_v, cidx_v, rows_v, sem, acc):
    cid = lax.axis_index("c")
    sid = lax.axis_index("s")
    wid = cid * NS + sid

    pltpu.sync_copy(zeros2_hbm, acc.at[pl.ds(sid * ROWS_PER_TILE,
                                             ROWS_PER_TILE)])
    plsc.subcore_barrier()

    pltpu.sync_copy(row2d_hbm.at[pl.ds(wid * CH_PER_TILE, CH_PER_TILE)], ridx_v)
    pltpu.sync_copy(col2d_hbm.at[pl.ds(wid * CH_PER_TILE, CH_PER_TILE)], cidx_v)

    def body(k, carry):
        pltpu.async_copy(hp_hbm.at[ridx_v.at[k]], rows_v, sem).wait()
        pltpu.sync_copy(rows_v, acc.at[cidx_v.at[k]], add=True)
        return carry
    lax.fori_loop(0, CH_PER_TILE, body, 0)
    plsc.subcore_barrier()

    base = sid * ROWS_PER_TILE
    pltpu.sync_copy(acc.at[pl.ds(base, ROWS_PER_TILE)],
                    part_hbm.at[cid, pl.ds(base, ROWS_PER_TILE)])


_spmm = functools.partial(
    pl.kernel, _spmm_body, mesh=_mesh,
    out_type=jax.ShapeDtypeStruct((NC, N_PAD, F), jnp.float32),
    scratch_types=[
        pltpu.VMEM((CH_PER_TILE, CHUNK), jnp.int32),
        pltpu.VMEM((CH_PER_TILE, CHUNK), jnp.int32),
        pltpu.VMEM((CHUNK, F), jnp.float32),
        pltpu.SemaphoreType.DMA,
        pltpu.VMEM_SHARED((N_PAD, F), jnp.float32),
    ],
)()


def _matmul_body(x_ref, w_ref, isd_ref, o_ref):
    o_ref[...] = jnp.dot(x_ref[...], w_ref[...],
                         preferred_element_type=jnp.float32) * isd_ref[...]


def _matmul_scaled(x, w, isd2d):
    blk = 1000
    return pl.pallas_call(
        _matmul_body,
        grid=(N_NODES // blk,),
        in_specs=[
            pl.BlockSpec((blk, F), lambda i: (i, 0)),
            pl.BlockSpec((F, F), lambda i: (0, 0)),
            pl.BlockSpec((blk, 1), lambda i: (i, 0)),
        ],
        out_specs=pl.BlockSpec((blk, F), lambda i: (i, 0)),
        out_shape=jax.ShapeDtypeStruct((N_NODES, F), jnp.float32),
    )(x, w, isd2d)


def _combine_body(p_ref, isd_ref, b_ref, o_ref):
    p = p_ref[...]
    o_ref[...] = (p[0] + p[1]) * isd_ref[...] + b_ref[...]


def _combine(partials, isd2d, b2d):
    blk = 500
    return pl.pallas_call(
        _combine_body,
        grid=(N_NODES // blk,),
        in_specs=[
            pl.BlockSpec((NC, blk, F), lambda i: (0, i, 0)),
            pl.BlockSpec((blk, 1), lambda i: (i, 0)),
            pl.BlockSpec((1, F), lambda i: (0, 0)),
        ],
        out_specs=pl.BlockSpec((blk, F), lambda i: (i, 0)),
        out_shape=jax.ShapeDtypeStruct((N_NODES, F), jnp.float32),
    )(partials, isd2d, b2d)


def kernel(input_feature, edge_index, W, b):
    row = edge_index[0]
    col = edge_index[1]
    pad = E_PAD - N_EDGES
    row2d = jnp.concatenate(
        [row, jnp.zeros((pad,), jnp.int32)]).reshape(E_PAD // CHUNK, CHUNK)
    col2d = jnp.concatenate(
        [col, jnp.full((pad,), DUMMY_COL, jnp.int32)]).reshape(
            E_PAD // CHUNK, CHUNK)
    zeros1 = jnp.zeros((ROWS_PER_TILE,), jnp.float32)
    zeros2 = jnp.zeros((ROWS_PER_TILE, F), jnp.float32)

    isd_pad = _deg_isd(col2d, zeros1)
    isd2d = isd_pad[:N_NODES].reshape(N_NODES, 1)
    hp = _matmul_scaled(input_feature, W, isd2d)
    partials = _spmm(row2d, col2d, hp, zeros2)
    return _combine(partials, isd2d, b.reshape(1, F))


# speedup vs baseline: 11.0029x; 11.0029x over previous
import functools

import jax
import jax.numpy as jnp
from jax import lax
from jax.experimental import pallas as pl
from jax.experimental.pallas import tpu as pltpu
from jax.experimental.pallas import tpu_sc as plsc

N_NODES = 10000
N_EDGES = 320000
F = 128

NC = 2
NS = 16
NW = NC * NS

CHUNK = 128
CH_PER_TILE = 80
E_PAD = NW * CH_PER_TILE * CHUNK
N_PAD = 10240
ROWS_PER_TILE = N_PAD // NS
DUMMY_COL = N_NODES

_mesh = plsc.VectorSubcoreMesh(
    core_axis_name="c", subcore_axis_name="s", num_cores=NC, num_subcores=NS)


def _deg_body(col2d_hbm, zeros1_hbm, deg_hbm, cidx_v, ones_v, deg_acc):
    cid = lax.axis_index("c")
    sid = lax.axis_index("s")

    @pl.when(cid == 0)
    def _():
        pltpu.sync_copy(zeros1_hbm, deg_acc.at[pl.ds(sid * ROWS_PER_TILE,
                                                     ROWS_PER_TILE)])
        for j in range(CHUNK // 16):
            ones_v[pl.ds(j * 16, 16)] = jnp.ones((16,), jnp.float32)
        plsc.subcore_barrier()

        rows = (E_PAD // CHUNK) // NS
        pltpu.sync_copy(col2d_hbm.at[pl.ds(sid * rows, rows)], cidx_v)

        def body(k, carry):
            pltpu.sync_copy(ones_v, deg_acc.at[cidx_v.at[k]], add=True)
            return carry
        lax.fori_loop(0, rows, body, 0)
        plsc.subcore_barrier()

        base = sid * ROWS_PER_TILE
        pltpu.sync_copy(deg_acc.at[pl.ds(base, ROWS_PER_TILE)],
                        deg_hbm.at[pl.ds(base, ROWS_PER_TILE)])


_deg = functools.partial(
    pl.kernel, _deg_body, mesh=_mesh,
    out_type=jax.ShapeDtypeStruct((N_PAD,), jnp.float32),
    scratch_types=[
        pltpu.VMEM(((E_PAD // CHUNK) // NS, CHUNK), jnp.int32),
        pltpu.VMEM((CHUNK,), jnp.float32),
        pltpu.VMEM_SHARED((N_PAD,), jnp.float32),
    ],
)()


def _spmm_body(row2d_hbm, col2d_hbm, hp_hbm, zeros2_hbm, part_hbm,
               ridx_v, cidx_v, rows_v, sem, acc):
    cid = lax.axis_index("c")
    sid = lax.axis_index("s")
    wid = cid * NS + sid

    pltpu.sync_copy(zeros2_hbm, acc.at[pl.ds(sid * ROWS_PER_TILE,
                                             ROWS_PER_TILE)])
    plsc.subcore_barrier()

    pltpu.sync_copy(row2d_hbm.at[pl.ds(wid * CH_PER_TILE, CH_PER_TILE)], ridx_v)
    pltpu.sync_copy(col2d_hbm.at[pl.ds(wid * CH_PER_TILE, CH_PER_TILE)], cidx_v)

    def body(k, carry):
        pltpu.async_copy(hp_hbm.at[ridx_v.at[k]], rows_v, sem).wait()
        pltpu.sync_copy(rows_v, acc.at[cidx_v.at[k]], add=True)
        return carry
    lax.fori_loop(0, CH_PER_TILE, body, 0)
    plsc.subcore_barrier()

    base = sid * ROWS_PER_TILE
    pltpu.sync_copy(acc.at[pl.ds(base, ROWS_PER_TILE)],
                    part_hbm.at[cid, pl.ds(base, ROWS_PER_TILE)])


_spmm = functools.partial(
    pl.kernel, _spmm_body, mesh=_mesh,
    out_type=jax.ShapeDtypeStruct((NC, N_PAD, F), jnp.float32),
    scratch_types=[
        pltpu.VMEM((CH_PER_TILE, CHUNK), jnp.int32),
        pltpu.VMEM((CH_PER_TILE, CHUNK), jnp.int32),
        pltpu.VMEM((CHUNK, F), jnp.float32),
        pltpu.SemaphoreType.DMA,
        pltpu.VMEM_SHARED((N_PAD, F), jnp.float32),
    ],
)()


def _matmul_body(x_ref, w_ref, deg_ref, o_ref):
    isd = lax.rsqrt(jnp.maximum(deg_ref[...], jnp.float32(1.0)))
    o_ref[...] = jnp.dot(x_ref[...], w_ref[...],
                         preferred_element_type=jnp.float32) * isd


def _matmul_scaled(x, w, deg2d):
    blk = 1000
    return pl.pallas_call(
        _matmul_body,
        grid=(N_NODES // blk,),
        in_specs=[
            pl.BlockSpec((blk, F), lambda i: (i, 0)),
            pl.BlockSpec((F, F), lambda i: (0, 0)),
            pl.BlockSpec((blk, 1), lambda i: (i, 0)),
        ],
        out_specs=pl.BlockSpec((blk, F), lambda i: (i, 0)),
        out_shape=jax.ShapeDtypeStruct((N_NODES, F), jnp.float32),
    )(x, w, deg2d)


def _combine_body(p_ref, deg_ref, b_ref, o_ref):
    p = p_ref[...]
    isd = lax.rsqrt(jnp.maximum(deg_ref[...], jnp.float32(1.0)))
    o_ref[...] = (p[0] + p[1]) * isd + b_ref[...]


def _combine(partials, deg2d, b2d):
    blk = 1000
    return pl.pallas_call(
        _combine_body,
        grid=(N_NODES // blk,),
        in_specs=[
            pl.BlockSpec((NC, blk, F), lambda i: (0, i, 0)),
            pl.BlockSpec((blk, 1), lambda i: (i, 0)),
            pl.BlockSpec((1, F), lambda i: (0, 0)),
        ],
        out_specs=pl.BlockSpec((blk, F), lambda i: (i, 0)),
        out_shape=jax.ShapeDtypeStruct((N_NODES, F), jnp.float32),
    )(partials, deg2d, b2d)


def kernel(input_feature, edge_index, W, b):
    row = edge_index[0]
    col = edge_index[1]
    pad = E_PAD - N_EDGES
    row2d = jnp.concatenate(
        [row, jnp.zeros((pad,), jnp.int32)]).reshape(E_PAD // CHUNK, CHUNK)
    col2d = jnp.concatenate(
        [col, jnp.full((pad,), DUMMY_COL, jnp.int32)]).reshape(
            E_PAD // CHUNK, CHUNK)
    zeros1 = jnp.zeros((ROWS_PER_TILE,), jnp.float32)
    zeros2 = jnp.zeros((ROWS_PER_TILE, F), jnp.float32)

    deg_pad = _deg(col2d, zeros1)
    deg2d = deg_pad[:N_NODES].reshape(N_NODES, 1)
    hp = _matmul_scaled(input_feature, W, deg2d)
    partials = _spmm(row2d, col2d, hp, zeros2)
    return _combine(partials, deg2d, b.reshape(1, F))


# speedup vs baseline: 12.1545x; 1.1047x over previous
import functools

import jax
import jax.numpy as jnp
from jax import lax
from jax.experimental import pallas as pl
from jax.experimental.pallas import tpu as pltpu
from jax.experimental.pallas import tpu_sc as plsc

N_NODES = 10000
N_EDGES = 320000
F = 128

NC = 2
NS = 16
NW = NC * NS

CHUNK = 128
CH_PER_TILE = 80
CH_HALF = 40
E_PAD = NW * CH_PER_TILE * CHUNK
N_PAD = 10240
ROWS_PER_TILE = N_PAD // NS
DUMMY_COL = N_NODES

_mesh = plsc.VectorSubcoreMesh(
    core_axis_name="c", subcore_axis_name="s", num_cores=NC, num_subcores=NS)


def _deg_body(col2d_hbm, zeros1_hbm, deg_hbm, cidx_v, ones_v, deg_acc):
    cid = lax.axis_index("c")
    sid = lax.axis_index("s")

    @pl.when(cid == 0)
    def _():
        pltpu.sync_copy(zeros1_hbm, deg_acc.at[pl.ds(sid * ROWS_PER_TILE,
                                                     ROWS_PER_TILE)])
        for j in range(CHUNK // 16):
            ones_v[pl.ds(j * 16, 16)] = jnp.ones((16,), jnp.float32)
        plsc.subcore_barrier()

        rows = (E_PAD // CHUNK) // NS
        pltpu.sync_copy(col2d_hbm.at[pl.ds(sid * rows, rows)], cidx_v)

        def body(k, carry):
            pltpu.sync_copy(ones_v, deg_acc.at[cidx_v.at[k]], add=True)
            return carry
        lax.fori_loop(0, rows, body, 0)
        plsc.subcore_barrier()

        base = sid * ROWS_PER_TILE
        pltpu.sync_copy(deg_acc.at[pl.ds(base, ROWS_PER_TILE)],
                        deg_hbm.at[pl.ds(base, ROWS_PER_TILE)])


_deg = functools.partial(
    pl.kernel, _deg_body, mesh=_mesh,
    out_type=jax.ShapeDtypeStruct((N_PAD,), jnp.float32),
    scratch_types=[
        pltpu.VMEM(((E_PAD // CHUNK) // NS, CHUNK), jnp.int32),
        pltpu.VMEM((CHUNK,), jnp.float32),
        pltpu.VMEM_SHARED((N_PAD,), jnp.float32),
    ],
)()


def _spmm_body(row2d_hbm, col2d_hbm, hp_hbm, zeros2_hbm, part_hbm,
               ridx_v, cidx_v, rows0_v, rows1_v, sem0, sem1, acc):
    cid = lax.axis_index("c")
    sid = lax.axis_index("s")
    wid = cid * NS + sid

    pltpu.sync_copy(zeros2_hbm, acc.at[pl.ds(sid * ROWS_PER_TILE,
                                             ROWS_PER_TILE)])
    plsc.subcore_barrier()

    def body(kk, carry):
        a = kk * 2
        pltpu.async_copy(hp_hbm.at[ridx_v.at[a + 1]], rows1_v, sem1)
        pltpu.make_async_copy(hp_hbm.at[ridx_v.at[a]], rows0_v, sem0).wait()
        pltpu.sync_copy(rows0_v, acc.at[cidx_v.at[a]], add=True)

        @pl.when(a + 2 < CH_HALF)
        def _():
            pltpu.async_copy(hp_hbm.at[ridx_v.at[a + 2]], rows0_v, sem0)
        pltpu.make_async_copy(hp_hbm.at[ridx_v.at[a + 1]], rows1_v, sem1).wait()
        pltpu.sync_copy(rows1_v, acc.at[cidx_v.at[a + 1]], add=True)
        return carry

    for h in range(CH_PER_TILE // CH_HALF):
        base = wid * CH_PER_TILE + h * CH_HALF
        pltpu.sync_copy(row2d_hbm.at[pl.ds(base, CH_HALF)], ridx_v)
        pltpu.sync_copy(col2d_hbm.at[pl.ds(base, CH_HALF)], cidx_v)
        pltpu.async_copy(hp_hbm.at[ridx_v.at[0]], rows0_v, sem0)
        lax.fori_loop(0, CH_HALF // 2, body, 0)
    plsc.subcore_barrier()

    base = sid * ROWS_PER_TILE
    pltpu.sync_copy(acc.at[pl.ds(base, ROWS_PER_TILE)],
                    part_hbm.at[cid, pl.ds(base, ROWS_PER_TILE)])


_spmm = functools.partial(
    pl.kernel, _spmm_body, mesh=_mesh,
    out_type=jax.ShapeDtypeStruct((NC, N_PAD, F), jnp.float32),
    scratch_types=[
        pltpu.VMEM((CH_HALF, CHUNK), jnp.int32),
        pltpu.VMEM((CH_HALF, CHUNK), jnp.int32),
        pltpu.VMEM((CHUNK, F), jnp.float32),
        pltpu.VMEM((CHUNK, F), jnp.float32),
        pltpu.SemaphoreType.DMA,
        pltpu.SemaphoreType.DMA,
        pltpu.VMEM_SHARED((N_PAD, F), jnp.float32),
    ],
)()


def _matmul_body(x_ref, w_ref, deg_ref, o_ref):
    isd = lax.rsqrt(jnp.maximum(deg_ref[...], jnp.float32(1.0)))
    o_ref[...] = jnp.dot(x_ref[...], w_ref[...],
                         preferred_element_type=jnp.float32) * isd


def _matmul_scaled(x, w, deg2d):
    blk = 1000
    return pl.pallas_call(
        _matmul_body,
        grid=(N_NODES // blk,),
        in_specs=[
            pl.BlockSpec((blk, F), lambda i: (i, 0)),
            pl.BlockSpec((F, F), lambda i: (0, 0)),
            pl.BlockSpec((blk, 1), lambda i: (i, 0)),
        ],
        out_specs=pl.BlockSpec((blk, F), lambda i: (i, 0)),
        out_shape=jax.ShapeDtypeStruct((N_NODES, F), jnp.float32),
    )(x, w, deg2d)


def _combine_body(p_ref, deg_ref, b_ref, o_ref):
    p = p_ref[...]
    isd = lax.rsqrt(jnp.maximum(deg_ref[...], jnp.float32(1.0)))
    o_ref[...] = (p[0] + p[1]) * isd + b_ref[...]


def _combine(partials, deg2d, b2d):
    blk = 1000
    return pl.pallas_call(
        _combine_body,
        grid=(N_NODES // blk,),
        in_specs=[
            pl.BlockSpec((NC, blk, F), lambda i: (0, i, 0)),
            pl.BlockSpec((blk, 1), lambda i: (i, 0)),
            pl.BlockSpec((1, F), lambda i: (0, 0)),
        ],
        out_specs=pl.BlockSpec((blk, F), lambda i: (i, 0)),
        out_shape=jax.ShapeDtypeStruct((N_NODES, F), jnp.float32),
    )(partials, deg2d, b2d)


def kernel(input_feature, edge_index, W, b):
    row = edge_index[0]
    col = edge_index[1]
    pad = E_PAD - N_EDGES
    row2d = jnp.concatenate(
        [row, jnp.zeros((pad,), jnp.int32)]).reshape(E_PAD // CHUNK, CHUNK)
    col2d = jnp.concatenate(
        [col, jnp.full((pad,), DUMMY_COL, jnp.int32)]).reshape(
            E_PAD // CHUNK, CHUNK)
    zeros1 = jnp.zeros((ROWS_PER_TILE,), jnp.float32)
    zeros2 = jnp.zeros((ROWS_PER_TILE, F), jnp.float32)

    deg_pad = _deg(col2d, zeros1)
    deg2d = deg_pad[:N_NODES].reshape(N_NODES, 1)
    hp = _matmul_scaled(input_feature, W, deg2d)
    partials = _spmm(row2d, col2d, hp, zeros2)
    return _combine(partials, deg2d, b.reshape(1, F))


# speedup vs baseline: 14.2360x; 1.1713x over previous
import functools

import jax
import jax.numpy as jnp
from jax import lax
from jax.experimental import pallas as pl
from jax.experimental.pallas import tpu as pltpu
from jax.experimental.pallas import tpu_sc as plsc

N_NODES = 10000
N_EDGES = 320000
F = 128

NC = 2
NS = 16
NW = NC * NS

CHUNK = 128
CH_PER_TILE = 80
CH_HALF = 40
E_PAD = NW * CH_PER_TILE * CHUNK
N_PAD = 10240
ROWS_PER_TILE = N_PAD // NS
DUMMY_COL = N_NODES

_mesh = plsc.VectorSubcoreMesh(
    core_axis_name="c", subcore_axis_name="s", num_cores=NC, num_subcores=NS)


def _deg_body(col2d_hbm, zeros1_hbm, deg_hbm, cidx_v, ones_v, deg_acc):
    cid = lax.axis_index("c")
    sid = lax.axis_index("s")

    @pl.when(cid == 0)
    def _():
        pltpu.sync_copy(zeros1_hbm, deg_acc.at[pl.ds(sid * ROWS_PER_TILE,
                                                     ROWS_PER_TILE)])
        for j in range(CHUNK // 16):
            ones_v[pl.ds(j * 16, 16)] = jnp.ones((16,), jnp.float32)
        plsc.subcore_barrier()

        rows = (E_PAD // CHUNK) // NS
        pltpu.sync_copy(col2d_hbm.at[pl.ds(sid * rows, rows)], cidx_v)

        def body(k, carry):
            pltpu.sync_copy(ones_v, deg_acc.at[cidx_v.at[k]], add=True)
            return carry
        lax.fori_loop(0, rows, body, 0)
        plsc.subcore_barrier()

        base = sid * ROWS_PER_TILE
        pltpu.sync_copy(deg_acc.at[pl.ds(base, ROWS_PER_TILE)],
                        deg_hbm.at[pl.ds(base, ROWS_PER_TILE)])


_deg = functools.partial(
    pl.kernel, _deg_body, mesh=_mesh,
    out_type=jax.ShapeDtypeStruct((N_PAD,), jnp.float32),
    scratch_types=[
        pltpu.VMEM(((E_PAD // CHUNK) // NS, CHUNK), jnp.int32),
        pltpu.VMEM((CHUNK,), jnp.float32),
        pltpu.VMEM_SHARED((N_PAD,), jnp.float32),
    ],
)()


def _spmm_body(row2d_hbm, col2d_hbm, hp_hbm, zeros2_hbm, part_hbm,
               ridx_v, cidx_v, rows0_v, rows1_v, sem0, sem1, acc):
    cid = lax.axis_index("c")
    sid = lax.axis_index("s")
    wid = cid * NS + sid

    pltpu.sync_copy(zeros2_hbm, acc.at[pl.ds(sid * ROWS_PER_TILE,
                                             ROWS_PER_TILE)])
    plsc.subcore_barrier()

    def body(kk, carry):
        a = kk * 2
        pltpu.async_copy(hp_hbm.at[ridx_v.at[a + 1]], rows1_v, sem1)
        pltpu.make_async_copy(hp_hbm.at[ridx_v.at[a]], rows0_v, sem0).wait()
        pltpu.sync_copy(rows0_v, acc.at[cidx_v.at[a]], add=True)

        @pl.when(a + 2 < CH_HALF)
        def _():
            pltpu.async_copy(hp_hbm.at[ridx_v.at[a + 2]], rows0_v, sem0)
        pltpu.make_async_copy(hp_hbm.at[ridx_v.at[a + 1]], rows1_v, sem1).wait()
        pltpu.sync_copy(rows1_v, acc.at[cidx_v.at[a + 1]], add=True)
        return carry

    for h in range(CH_PER_TILE // CH_HALF):
        base = wid * CH_PER_TILE + h * CH_HALF
        pltpu.sync_copy(row2d_hbm.at[pl.ds(base, CH_HALF)], ridx_v)
        pltpu.sync_copy(col2d_hbm.at[pl.ds(base, CH_HALF)], cidx_v)
        pltpu.async_copy(hp_hbm.at[ridx_v.at[0]], rows0_v, sem0)
        lax.fori_loop(0, CH_HALF // 2, body, 0)
    plsc.subcore_barrier()

    base = sid * ROWS_PER_TILE
    pltpu.sync_copy(acc.at[pl.ds(base, ROWS_PER_TILE)],
                    part_hbm.at[cid, pl.ds(base, ROWS_PER_TILE)])


_spmm = functools.partial(
    pl.kernel, _spmm_body, mesh=_mesh,
    out_type=jax.ShapeDtypeStruct((NC, N_PAD, F), jnp.float32),
    scratch_types=[
        pltpu.VMEM((CH_HALF, CHUNK), jnp.int32),
        pltpu.VMEM((CH_HALF, CHUNK), jnp.int32),
        pltpu.VMEM((CHUNK, F), jnp.float32),
        pltpu.VMEM((CHUNK, F), jnp.float32),
        pltpu.SemaphoreType.DMA,
        pltpu.SemaphoreType.DMA,
        pltpu.VMEM_SHARED((N_PAD, F), jnp.float32),
    ],
)()


def _matmul_body(x_ref, w_ref, deg_ref, o_ref):
    isd = lax.rsqrt(jnp.maximum(deg_ref[...], jnp.float32(1.0)))
    o_ref[...] = jnp.dot(x_ref[...], w_ref[...],
                         preferred_element_type=jnp.float32) * isd


def _matmul_scaled(x, w, deg2d):
    blk = 1000
    return pl.pallas_call(
        _matmul_body,
        grid=(N_NODES // blk,),
        in_specs=[
            pl.BlockSpec((blk, F), lambda i: (i, 0)),
            pl.BlockSpec((F, F), lambda i: (0, 0)),
            pl.BlockSpec((blk, 1), lambda i: (i, 0)),
        ],
        out_specs=pl.BlockSpec((blk, F), lambda i: (i, 0)),
        out_shape=jax.ShapeDtypeStruct((N_NODES, F), jnp.float32),
    )(x, w, deg2d)


def _combine_body(p_ref, deg_ref, b_ref, o_ref):
    p = p_ref[...]
    isd = lax.rsqrt(jnp.maximum(deg_ref[...], jnp.float32(1.0)))
    o_ref[...] = (p[0] + p[1]) * isd + b_ref[...]


def _combine(partials, deg2d, b2d):
    blk = 1000
    return pl.pallas_call(
        _combine_body,
        grid=(N_NODES // blk,),
        in_specs=[
            pl.BlockSpec((NC, blk, F), lambda i: (0, i, 0)),
            pl.BlockSpec((blk, 1), lambda i: (i, 0)),
            pl.BlockSpec((1, F), lambda i: (0, 0)),
        ],
        out_specs=pl.BlockSpec((blk, F), lambda i: (i, 0)),
        out_shape=jax.ShapeDtypeStruct((N_NODES, F), jnp.float32),
    )(partials, deg2d, b2d)


def kernel(input_feature, edge_index, W, b):
    row = edge_index[0]
    col = edge_index[1]
    pad_t = (E_PAD - N_EDGES) // NW
    real_t = N_EDGES // NW
    dummy_cols = jnp.broadcast_to(
        DUMMY_COL + jnp.arange(pad_t, dtype=jnp.int32), (NW, pad_t))
    row2d = jnp.concatenate(
        [row.reshape(NW, real_t),
         jnp.zeros((NW, pad_t), jnp.int32)], axis=1).reshape(
            E_PAD // CHUNK, CHUNK)
    col2d = jnp.concatenate(
        [col.reshape(NW, real_t), dummy_cols], axis=1).reshape(
            E_PAD // CHUNK, CHUNK)
    zeros1 = jnp.zeros((ROWS_PER_TILE,), jnp.float32)
    zeros2 = jnp.zeros((ROWS_PER_TILE, F), jnp.float32)

    deg_pad = _deg(col2d, zeros1)
    deg2d = deg_pad[:N_NODES].reshape(N_NODES, 1)
    hp = _matmul_scaled(input_feature, W, deg2d)
    partials = _spmm(row2d, col2d, hp, zeros2)
    return _combine(partials, deg2d, b.reshape(1, F))
